# Initial kernel scaffold; baseline (speedup 1.0000x reference)
import jax
import jax.numpy as jnp
from jax.experimental import pallas as pl


def kernel(x, edge_index, edge_weight, W):
    raise NotImplementedError("write your pallas kernel here")


# speedup vs baseline: 10.1998x; 10.1998x over previous
import functools

import jax
import jax.numpy as jnp
from jax import lax
from jax.experimental import pallas as pl
from jax.experimental.pallas import tpu as pltpu
from jax.experimental.pallas import tpu_sc as plsc

N_NODES = 10000
N_EDGES = 320000
D_FEAT = 128
D_OUT = 128

NC = 2
NS = 16
NW = NC * NS

N_PAD = 10240
STRIPE = N_PAD // NS

EPW = N_EDGES // NW
CH = 80
NCH = EPW // CH

_sc_mesh = plsc.VectorSubcoreMesh(
    core_axis_name="c", subcore_axis_name="s", num_cores=NC, num_subcores=NS
)


def _sc_body(src_hbm, dst_hbm, w_hbm, x_hbm, np_hbm, deg_hbm,
             src_all, dst_all, w_all, src_ch, dst_ch, rows, dtmp,
             acc_sh, deg_sh):
  c = lax.axis_index("c")
  s = lax.axis_index("s")
  wid = s * NC + c

  def _zrow(r, _):
    for j in range(D_FEAT // 16):
      rows[r, pl.ds(16 * j, 16)] = jnp.zeros((16,), jnp.float32)
    return 0
  lax.fori_loop(0, CH, _zrow, 0)
  for k in range(CH // 16):
    dtmp[pl.ds(16 * k, 16)] = jnp.zeros((16,), jnp.float32)
  for k in range(STRIPE // CH):
    r0 = s * STRIPE + k * CH
    pltpu.sync_copy(rows, acc_sh.at[pl.ds(r0, CH)])
    pltpu.sync_copy(dtmp, deg_sh.at[pl.ds(r0, CH)])
  plsc.subcore_barrier()

  base = wid * EPW
  pltpu.sync_copy(src_hbm.at[pl.ds(base, EPW)], src_all)
  pltpu.sync_copy(dst_hbm.at[pl.ds(base, EPW)], dst_all)
  pltpu.sync_copy(w_hbm.at[pl.ds(base, EPW)], w_all)

  def _chunk(i, _):
    off = i * CH
    for k in range(CH // 16):
      src_ch[pl.ds(16 * k, 16)] = src_all[pl.ds(off + 16 * k, 16)]
      dst_ch[pl.ds(16 * k, 16)] = dst_all[pl.ds(off + 16 * k, 16)]
    pltpu.sync_copy(x_hbm.at[src_ch], rows)

    def _scale(k, _):
      e0 = 16 * k
      w16 = w_all[pl.ds(off + e0, 16)]
      for l in range(16):
        wv = w16[l]
        for j in range(D_FEAT // 16):
          sl = pl.ds(16 * j, 16)
          rows[e0 + l, sl] = rows[e0 + l, sl] * wv
      return 0
    lax.fori_loop(0, CH // 16, _scale, 0)

    pltpu.sync_copy(rows, acc_sh.at[dst_ch], add=True)
    pltpu.sync_copy(w_all.at[pl.ds(off, CH)], deg_sh.at[dst_ch], add=True)
    return 0
  lax.fori_loop(0, NCH, _chunk, 0)
  plsc.subcore_barrier()

  for k in range(STRIPE // CH):
    r0 = s * STRIPE + k * CH
    pltpu.sync_copy(acc_sh.at[pl.ds(r0, CH)], rows)
    pltpu.sync_copy(rows, np_hbm.at[c, pl.ds(r0, CH)])
    pltpu.sync_copy(deg_sh.at[pl.ds(r0, CH)], dtmp)
    pltpu.sync_copy(dtmp, deg_hbm.at[pl.ds(c * N_PAD + r0, CH)])


_sc_call = pl.kernel(
    _sc_body,
    out_type=(
        jax.ShapeDtypeStruct((NC, N_PAD, D_FEAT), jnp.float32),
        jax.ShapeDtypeStruct((NC * N_PAD,), jnp.float32),
    ),
    mesh=_sc_mesh,
    scratch_types=(
        pltpu.VMEM((EPW,), jnp.int32),
        pltpu.VMEM((EPW,), jnp.int32),
        pltpu.VMEM((EPW,), jnp.float32),
        pltpu.VMEM((CH,), jnp.int32),
        pltpu.VMEM((CH,), jnp.int32),
        pltpu.VMEM((CH, D_FEAT), jnp.float32),
        pltpu.VMEM((CH,), jnp.float32),
        pltpu.VMEM_SHARED((N_PAD, D_FEAT), jnp.float32),
        pltpu.VMEM_SHARED((N_PAD,), jnp.float32),
    ),
)


_TC_R = 1000


def _tc_body(x_ref, p0_ref, p1_ref, d0_ref, d1_ref, w1_ref, w2_ref, o_ref):
  d = d0_ref[...] + d1_ref[...]
  neigh = (p0_ref[...] + p1_ref[...]) / (d + 1e-9)
  acc = jnp.dot(x_ref[...], w1_ref[...], preferred_element_type=jnp.float32)
  acc = acc + jnp.dot(neigh, w2_ref[...], preferred_element_type=jnp.float32)
  o_ref[...] = acc * jax.nn.sigmoid(acc)


_tc_call = pl.pallas_call(
    _tc_body,
    grid=(N_NODES // _TC_R,),
    in_specs=[
        pl.BlockSpec((_TC_R, D_FEAT), lambda i: (i, 0)),
        pl.BlockSpec((_TC_R, D_FEAT), lambda i: (i, 0)),
        pl.BlockSpec((_TC_R, D_FEAT), lambda i: (i, 0)),
        pl.BlockSpec((_TC_R, 1), lambda i: (i, 0)),
        pl.BlockSpec((_TC_R, 1), lambda i: (i, 0)),
        pl.BlockSpec((D_FEAT, D_OUT), lambda i: (0, 0)),
        pl.BlockSpec((D_FEAT, D_OUT), lambda i: (0, 0)),
    ],
    out_specs=pl.BlockSpec((_TC_R, D_OUT), lambda i: (i, 0)),
    out_shape=jax.ShapeDtypeStruct((N_NODES, D_OUT), jnp.float32),
)


@jax.jit
def kernel(x, edge_index, edge_weight, W):
  src = edge_index[0].astype(jnp.int32)
  dst = edge_index[1].astype(jnp.int32)
  w = edge_weight.astype(jnp.float32)
  np_out, deg_out = _sc_call(src, dst, w, x)
  p0 = np_out[0, :N_NODES]
  p1 = np_out[1, :N_NODES]
  d0 = deg_out[:N_NODES].reshape(N_NODES, 1)
  d1 = deg_out[N_PAD:N_PAD + N_NODES].reshape(N_NODES, 1)
  return _tc_call(x, p0, p1, d0, d1, W[:D_FEAT], W[D_FEAT:])
